# baseline (speedup 1.0000x reference)
import jax
import jax.numpy as jnp
from jax.experimental import pallas as pl

_N = 50000
_D = 128
_M = 12500
_R = 0.1
_K = 16


def _fps(pos, m):
    d0 = jnp.sum((pos - pos[0]) ** 2, axis=1)
    sel0 = jnp.zeros((m,), jnp.int32)

    def body(i, state):
        dists, sel = state
        last = sel[i - 1]
        d = jnp.sum((pos - pos[last]) ** 2, axis=1)
        dists = jnp.minimum(dists, d)
        nxt = jnp.argmax(dists).astype(jnp.int32)
        return dists, sel.at[i].set(nxt)

    _, sel = jax.lax.fori_loop(1, m, body, (d0, sel0))
    return sel


def _radius_knn(pos_all, pos_q, r, k, chunk=500):
    nq = pos_q.shape[0]
    qc = pos_q.reshape(nq // chunk, chunk, 3)

    def per_chunk(q):
        d = jnp.sum((q[:, None, :] - pos_all[None, :, :]) ** 2, axis=-1)
        within = d <= r * r
        neg = jnp.where(within, -d, -jnp.inf)
        vals, idxs = jax.lax.top_k(neg, k)
        return idxs.astype(jnp.int32), vals > -jnp.inf

    idxs, mask = jax.lax.map(per_chunk, qc)
    return idxs.reshape(nq, k), mask.reshape(nq, k)


def _ln_body(x_ref, g_ref, b_ref, o_ref):
    x = x_ref[...]
    mu = jnp.mean(x, axis=-1, keepdims=True)
    var = jnp.mean((x - mu) ** 2, axis=-1, keepdims=True)
    o_ref[...] = (x - mu) / jnp.sqrt(var + 1e-5) * g_ref[...] + b_ref[...]


def _layernorm(x, g, b):
    m, d = x.shape
    return pl.pallas_call(
        _ln_body,
        out_shape=jax.ShapeDtypeStruct((m, d), x.dtype),
    )(x, g.reshape(1, d), b.reshape(1, d))


def kernel(x, pos, batch, W_lin, b_lin, W_src, b_src, W_dst, b_dst,
           W_p1, b_p1, W_p2, b_p2, W_a, b_a, ln_g, ln_b):
    sel = _fps(pos, _M)
    pos_q = pos[sel]
    nbr, mask = _radius_knn(pos, pos_q, _R, _K)
    v = x @ W_lin + b_lin
    a_src = x @ W_src + b_src
    a_dst = x[sel] @ W_dst + b_dst
    xj = v[nbr]
    rel = pos_q[:, None, :] - pos[nbr]
    delta = jax.nn.relu(jax.nn.relu(rel @ W_p1 + b_p1) @ W_p2 + b_p2)
    alpha = a_dst[:, None, :] - a_src[nbr] + delta
    alpha = jax.nn.relu(alpha @ W_a + b_a)
    alpha = jnp.where(mask[:, :, None], alpha, -jnp.inf)
    alpha = jax.nn.softmax(alpha, axis=1)
    out = jnp.sum(alpha * (xj + delta), axis=1)
    out = _layernorm(out, ln_g, ln_b)
    return out, pos_q, batch[sel]


# speedup vs baseline: 5.1541x; 5.1541x over previous
import functools

import jax
import jax.numpy as jnp
from jax.experimental import pallas as pl
from jax.experimental.pallas import tpu as pltpu

_N = 50000
_D = 128
_M = 12500
_R = 0.1
_K = 16

_ROWS = 8
_IMAXV = 2**31 - 1


def _fps_body(m, rows, w, wp, planes_ref, lin_ref, pos0_ref, sel_ref,
              qx_ref, qy_ref, qz_ref, dists_ref):
    lin = lin_ref[...]
    dists_ref[...] = jnp.where(lin < jnp.int32(rows * w),
                               jnp.inf, -jnp.inf).astype(jnp.float32)
    lane128 = jax.lax.broadcasted_iota(jnp.int32, (1, 128), 1)
    lanew = jax.lax.broadcasted_iota(jnp.int32, (1, wp), 1)

    def body(i, carry):
        nxt, wx, wy, wz, a_s, a_x, a_y, a_z = carry
        laneq = lane128 == (i % 128)
        a_s = jnp.where(laneq, nxt, a_s)
        a_x = jnp.where(laneq, wx, a_x)
        a_y = jnp.where(laneq, wy, a_y)
        a_z = jnp.where(laneq, wz, a_z)
        blk = i // 128

        @pl.when((i % 128 == 127) | (i == m - 1))
        def _():
            sel_ref[pl.ds(blk, 1), :] = a_s
            qx_ref[pl.ds(blk, 1), :] = a_x
            qy_ref[pl.ds(blk, 1), :] = a_y
            qz_ref[pl.ds(blk, 1), :] = a_z

        dx = planes_ref[0] - wx
        dy = planes_ref[1] - wy
        dz = planes_ref[2] - wz
        d = (dx * dx + dz * dz) + dy * dy
        nd = jnp.minimum(dists_ref[...], d)
        dists_ref[...] = nd
        mx = jnp.max(nd)
        nxt2 = jnp.min(jnp.where(nd == mx, lin, jnp.int32(_IMAXV)))
        r = nxt2 // w
        c = nxt2 % w
        lmask = lanew == c
        wx2 = jnp.sum(jnp.where(lmask, planes_ref[0, pl.ds(r, 1), :], 0.0))
        wy2 = jnp.sum(jnp.where(lmask, planes_ref[1, pl.ds(r, 1), :], 0.0))
        wz2 = jnp.sum(jnp.where(lmask, planes_ref[2, pl.ds(r, 1), :], 0.0))
        return (nxt2, wx2, wy2, wz2, a_s, a_x, a_y, a_z)

    zf = jnp.zeros((1, 128), jnp.float32)
    zi = jnp.zeros((1, 128), jnp.int32)
    init = (jnp.int32(0), pos0_ref[0, 0], pos0_ref[0, 1], pos0_ref[0, 2],
            zi, zf, zf, zf)
    jax.lax.fori_loop(0, m, body, init)


def _fps_pallas(pos, m, interpret=False):
    n = pos.shape[0]
    rows = _ROWS
    w = n // rows
    wp = ((w + 127) // 128) * 128
    nb = (m + 127) // 128
    planes = pos.T.reshape(3, rows, w)
    if wp > w:
        planes = jnp.pad(planes, ((0, 0), (0, 0), (0, wp - w)))
    r_iota = jax.lax.broadcasted_iota(jnp.int32, (rows, wp), 0)
    c_iota = jax.lax.broadcasted_iota(jnp.int32, (rows, wp), 1)
    lin = jnp.where(c_iota < w, r_iota * w + c_iota, jnp.int32(_IMAXV))
    pos0 = pos[0:1, :]

    out = pl.pallas_call(
        functools.partial(_fps_body, m, rows, w, wp),
        grid=(1,),
        in_specs=[
            pl.BlockSpec((3, rows, wp), lambda i: (0, 0, 0)),
            pl.BlockSpec((rows, wp), lambda i: (0, 0)),
            pl.BlockSpec(memory_space=pltpu.SMEM),
        ],
        out_specs=[
            pl.BlockSpec((nb, 128), lambda i: (0, 0)),
            pl.BlockSpec((nb, 128), lambda i: (0, 0)),
            pl.BlockSpec((nb, 128), lambda i: (0, 0)),
            pl.BlockSpec((nb, 128), lambda i: (0, 0)),
        ],
        out_shape=[
            jax.ShapeDtypeStruct((nb, 128), jnp.int32),
            jax.ShapeDtypeStruct((nb, 128), jnp.float32),
            jax.ShapeDtypeStruct((nb, 128), jnp.float32),
            jax.ShapeDtypeStruct((nb, 128), jnp.float32),
        ],
        scratch_shapes=[pltpu.VMEM((rows, wp), jnp.float32)],
        interpret=interpret,
    )(planes, lin, pos0)
    sel = out[0].reshape(-1)[:m]
    pos_q = jnp.stack([o.reshape(-1)[:m] for o in out[1:]], axis=-1)
    return sel, pos_q


def _radius_knn(pos_all, pos_q, r, k, chunk=500):
    nq = pos_q.shape[0]
    qc = pos_q.reshape(nq // chunk, chunk, 3)

    def per_chunk(q):
        d = jnp.sum((q[:, None, :] - pos_all[None, :, :]) ** 2, axis=-1)
        within = d <= r * r
        neg = jnp.where(within, -d, -jnp.inf)
        vals, idxs = jax.lax.top_k(neg, k)
        return idxs.astype(jnp.int32), vals > -jnp.inf

    idxs, mask = jax.lax.map(per_chunk, qc)
    return idxs.reshape(nq, k), mask.reshape(nq, k)


def _ln_body(x_ref, g_ref, b_ref, o_ref):
    x = x_ref[...]
    mu = jnp.mean(x, axis=-1, keepdims=True)
    var = jnp.mean((x - mu) ** 2, axis=-1, keepdims=True)
    o_ref[...] = (x - mu) / jnp.sqrt(var + 1e-5) * g_ref[...] + b_ref[...]


def _layernorm(x, g, b):
    m, d = x.shape
    return pl.pallas_call(
        _ln_body,
        out_shape=jax.ShapeDtypeStruct((m, d), x.dtype),
    )(x, g.reshape(1, d), b.reshape(1, d))


def kernel(x, pos, batch, W_lin, b_lin, W_src, b_src, W_dst, b_dst,
           W_p1, b_p1, W_p2, b_p2, W_a, b_a, ln_g, ln_b):
    sel, pos_q = _fps_pallas(pos, _M)
    nbr, mask = _radius_knn(pos, pos_q, _R, _K)
    v = x @ W_lin + b_lin
    a_src = x @ W_src + b_src
    a_dst = x[sel] @ W_dst + b_dst
    xj = v[nbr]
    rel = pos_q[:, None, :] - pos[nbr]
    delta = jax.nn.relu(jax.nn.relu(rel @ W_p1 + b_p1) @ W_p2 + b_p2)
    alpha = a_dst[:, None, :] - a_src[nbr] + delta
    alpha = jax.nn.relu(alpha @ W_a + b_a)
    alpha = jnp.where(mask[:, :, None], alpha, -jnp.inf)
    alpha = jax.nn.softmax(alpha, axis=1)
    out = jnp.sum(alpha * (xj + delta), axis=1)
    out = _layernorm(out, ln_g, ln_b)
    return out, pos_q, batch[sel]
